# Initial kernel scaffold; baseline (speedup 1.0000x reference)
import jax
import jax.numpy as jnp
from jax.experimental import pallas as pl


def kernel(in_feat, edge_index, W1, b1, W2, b2, W3, b3, W4, b4, W5, b5, W6, b6, W7, b7):
    raise NotImplementedError("write your pallas kernel here")


# speedup vs baseline: 8.3834x; 8.3834x over previous
import functools

import jax
import jax.numpy as jnp
from jax import lax
from jax.experimental import pallas as pl
from jax.experimental.pallas import tpu as pltpu
from jax.experimental.pallas import tpu_sc as plsc

N = 10000
NP = 10240
E = 320000
CH = 125
NSUB = 16
ROWS_PER_SUB = NP // NSUB
ZR = 128

_F32 = jnp.float32


def _make_sc_agg(F, csub, a_s, b_s, a_d, b_d, gather, n_pass=1):
    mesh = plsc.VectorSubcoreMesh(core_axis_name="c", subcore_axis_name="s")
    csub_p = csub // n_pass

    def body(*refs):
        if gather:
            (sidx_hbm, didx_hbm, tab_hbm, out_hbm,
             sidx_v, didx_v, rows_v, acc, sem) = refs
        else:
            (didx_hbm, out_hbm, didx_v, rows_v, acc, sem) = refs
        cid = lax.axis_index("c")
        sid = lax.axis_index("s")

        def _fill(val):
            def _row(i, c):
                for j in range(F // 16):
                    rows_v[i, pl.ds(j * 16, 16)] = jnp.full((16,), val, _F32)
                return c
            lax.fori_loop(0, CH, _row, 0)
        _fill(0.0)
        base = sid * ROWS_PER_SUB
        for r in range(ROWS_PER_SUB // CH):
            pltpu.sync_copy(rows_v, acc.at[pl.ds(base + r * CH, CH)])
        rem = ROWS_PER_SUB % CH
        if rem:
            pltpu.sync_copy(rows_v.at[pl.ds(0, rem)],
                            acc.at[pl.ds(base + (ROWS_PER_SUB // CH) * CH, rem)])
        if not gather:
            _fill(1.0)
        plsc.subcore_barrier()

        for p in range(n_pass):
            if gather:
                pltpu.sync_copy(
                    sidx_hbm.at[pl.ds(cid * a_s + sid * b_s + p * csub_p,
                                      csub_p)], sidx_v)
            pltpu.sync_copy(
                didx_hbm.at[pl.ds(cid * a_d + sid * b_d + p * csub_p,
                                  csub_p)], didx_v)

            def _chunk(k, c):
                if gather:
                    pltpu.async_copy(tab_hbm.at[sidx_v.at[k]], rows_v,
                                     sem).wait()
                pltpu.sync_copy(rows_v, acc.at[didx_v.at[k]], add=True)
                return c
            lax.fori_loop(0, csub_p, _chunk, 0)
        plsc.subcore_barrier()

        pltpu.sync_copy(
            acc.at[pl.ds(sid * ROWS_PER_SUB, ROWS_PER_SUB)],
            out_hbm.at[pl.ds(cid * NP + sid * ROWS_PER_SUB, ROWS_PER_SUB)])

    scratch = [
        pltpu.VMEM((csub_p, CH), jnp.int32),
        pltpu.VMEM((CH, F), _F32),
        pltpu.VMEM_SHARED((NP, F), _F32),
        pltpu.SemaphoreType.DMA,
    ]
    if gather:
        scratch.insert(0, pltpu.VMEM((csub_p, CH), jnp.int32))
    return pl.kernel(
        body,
        mesh=mesh,
        out_type=jax.ShapeDtypeStruct((2 * NP, F), _F32),
        scratch_types=scratch,
    )


_sc_degrees = _make_sc_agg(128, 160, 0, 0, 2560, 160, gather=False)
_sc_agg128 = _make_sc_agg(128, 80, 1280, 80, 1280, 80, gather=True)
_sc_agg256 = _make_sc_agg(128, 160, 2560, 160, 0, 160, gather=True, n_pass=2)


R = 2048
GRID = NP // R
_HI = jax.lax.Precision.HIGHEST


def _dot(x, w):
    return jnp.dot(x, w, precision=_HI, preferred_element_type=_F32)


def _b_rows(spec_rows):
    return pl.BlockSpec((spec_rows, 1), lambda i: (i, 0))


_B_P128 = pl.BlockSpec((2, R, 128), lambda i: (0, i, 0))
_B_ROWS128 = pl.BlockSpec((R, 128), lambda i: (i, 0))


def _tc_call(body, in_specs, out_specs, out_shape):
    return pl.pallas_call(
        body, grid=(GRID,), in_specs=in_specs, out_specs=out_specs,
        out_shape=out_shape)


def _rsqrt_body(x_ref, o_ref):
    o_ref[...] = lax.rsqrt(jnp.maximum(x_ref[:, 0:1], 1.0))


def _mm1_body(x_ref, do_ref, w_ref, o_ref):
    o_ref[...] = _dot(do_ref[...] * x_ref[...], w_ref[...])


def _ew2_body(p_ref, di_ref, do_ref, b_ref, o_ref):
    u = p_ref[0] + p_ref[1]
    o_ref[...] = do_ref[...] * jax.nn.relu(di_ref[...] * u + b_ref[...])


def _mm23_body(p_ref, di_ref, do_ref, b_ref, w2_ref, w3a_ref, w3b_ref, o_ref):
    u = p_ref[0] + p_ref[1]
    h = jax.nn.relu(_dot(di_ref[...] * u, w2_ref[...]) + b_ref[...])
    x = do_ref[...] * h
    o_ref[0] = _dot(x, w3a_ref[...])
    o_ref[1] = _dot(x, w3b_ref[...])


def _mm4_body(u_ref, di_ref, do_ref, ba_ref, bb_ref, wa_ref, wb_ref, o_ref):
    xa = do_ref[...] * jax.nn.relu(di_ref[...] * u_ref[0] + ba_ref[...])
    xb = do_ref[...] * jax.nn.relu(di_ref[...] * u_ref[1] + bb_ref[...])
    o_ref[...] = _dot(xa, wa_ref[...]) + _dot(xb, wb_ref[...])


def _mm_body(p_ref, di_ref, do_ref, b_ref, w_ref, o_ref):
    u = p_ref[0] + p_ref[1]
    x = do_ref[...] * jax.nn.relu(di_ref[...] * u + b_ref[...])
    o_ref[...] = _dot(x, w_ref[...])


def _final_body(p_ref, di_ref, w_ref, b_ref, o_ref):
    u = p_ref[0] + p_ref[1]
    o_ref[...] = _dot(di_ref[...] * u, w_ref[...]) + b_ref[...]


def kernel(in_feat, edge_index, W1, b1, W2, b2, W3, b3, W4, b4, W5, b5,
           W6, b6, W7, b7):
    src = edge_index[0]
    dst = edge_index[1]
    src_ch = src.reshape(E // CH, CH)
    dst_ch = dst.reshape(E // CH, CH)
    srcdst_ch = jnp.concatenate([src_ch, dst_ch], axis=0)
    srcp_ch = jnp.concatenate([src_ch, src_ch + NP], axis=0)

    x0 = jnp.pad(in_feat, ((0, NP - N), (0, 0)))

    deg2 = _sc_degrees(srcdst_ch)
    d_all = pl.pallas_call(
        _rsqrt_body, out_shape=jax.ShapeDtypeStruct((2 * NP, 1), _F32))(deg2)
    d_o = d_all[:NP]
    d_i = d_all[NP:]

    di_spec = _b_rows(R)
    do_spec = _b_rows(R)
    b128 = pl.BlockSpec((1, 128), lambda i: (0, 0))
    w128 = pl.BlockSpec((128, 128), lambda i: (0, 0))

    v1 = _tc_call(_mm1_body,
                  [_B_ROWS128, do_spec, w128],
                  _B_ROWS128, jax.ShapeDtypeStruct((NP, 128), _F32))(
                      x0, d_o, W1)
    u1 = _sc_agg128(src_ch, dst_ch, v1).reshape(2, NP, 128)

    t1 = _tc_call(_ew2_body,
                  [_B_P128, di_spec, do_spec, b128],
                  _B_ROWS128, jax.ShapeDtypeStruct((NP, 128), _F32))(
                      u1, d_i, d_o, b1.reshape(1, 128))
    u2 = _sc_agg128(src_ch, dst_ch, t1).reshape(2, NP, 128)

    w256 = pl.BlockSpec((128, 256), lambda i: (0, 0))
    w256_128 = pl.BlockSpec((256, 128), lambda i: (0, 0))
    v3 = _tc_call(_mm23_body,
                  [_B_P128, di_spec, do_spec,
                   pl.BlockSpec((1, 256), lambda i: (0, 0)),
                   w256, w256_128, w256_128],
                  _B_P128, jax.ShapeDtypeStruct((2, NP, 128), _F32))(
                      u2, d_i, d_o, b2.reshape(1, 256), W2,
                      W3[:, :128], W3[:, 128:])
    u3 = _sc_agg256(srcp_ch, dst_ch, v3.reshape(2 * NP, 128)).reshape(
        2, NP, 128)

    v4 = _tc_call(_mm4_body,
                  [_B_P128, di_spec, do_spec, b128, b128, w128, w128],
                  _B_ROWS128, jax.ShapeDtypeStruct((NP, 128), _F32))(
                      u3, d_i, d_o, b3[:128].reshape(1, 128),
                      b3[128:].reshape(1, 128), W4[:128], W4[128:])
    u4 = _sc_agg128(src_ch, dst_ch, v4).reshape(2, NP, 128)

    mm = functools.partial(
        _tc_call, _mm_body,
        [_B_P128, di_spec, do_spec, b128, w128])
    v5 = mm(_B_ROWS128, jax.ShapeDtypeStruct((NP, 128), _F32))(
        u4, d_i, d_o, b4.reshape(1, 128), W5)
    u5 = _sc_agg128(src_ch, dst_ch, v5).reshape(2, NP, 128)
    v6 = mm(_B_ROWS128, jax.ShapeDtypeStruct((NP, 128), _F32))(
        u5, d_i, d_o, b5.reshape(1, 128), W6)
    u6 = _sc_agg128(src_ch, dst_ch, v6).reshape(2, NP, 128)

    t6 = _tc_call(_ew2_body,
                  [_B_P128, di_spec, do_spec, b128],
                  _B_ROWS128, jax.ShapeDtypeStruct((NP, 128), _F32))(
                      u6, d_i, d_o, b6.reshape(1, 128))
    u7 = _sc_agg128(src_ch, dst_ch, t6).reshape(2, NP, 128)

    out = _tc_call(_final_body,
                   [_B_P128, di_spec,
                    pl.BlockSpec((128, 1), lambda i: (0, 0)),
                    pl.BlockSpec((1, 1), lambda i: (0, 0))],
                   pl.BlockSpec((R, 1), lambda i: (i, 0)),
                   jax.ShapeDtypeStruct((NP, 1), _F32))(
                       u7, d_i, W7, b7.reshape(1, 1))
    return out[:N]


# speedup vs baseline: 9.9387x; 1.1855x over previous
import functools

import jax
import jax.numpy as jnp
from jax import lax
from jax.experimental import pallas as pl
from jax.experimental.pallas import tpu as pltpu
from jax.experimental.pallas import tpu_sc as plsc

N = 10000
NP = 10240
E = 320000
CH = 125
NSUB = 16
ROWS_PER_SUB = NP // NSUB
ZR = 128

_F32 = jnp.float32


def _make_sc_agg(F, csub, a_s, b_s, a_d, b_d, gather, n_pass=1):
    mesh = plsc.VectorSubcoreMesh(core_axis_name="c", subcore_axis_name="s")
    csub_p = csub // n_pass

    npairs = csub_p // 2

    def body(*refs):
        if gather:
            (sidx_hbm, didx_hbm, tab_hbm, out_hbm,
             sidx_v, didx_v, rows_a, rows_b, acc,
             semga, semgb, semsa, semsb) = refs
        else:
            (didx_hbm, out_hbm, didx_v, rows_a, acc, semsa) = refs
        cid = lax.axis_index("c")
        sid = lax.axis_index("s")

        def _fill(val):
            def _row(i, c):
                for j in range(F // 16):
                    rows_a[i, pl.ds(j * 16, 16)] = jnp.full((16,), val, _F32)
                return c
            lax.fori_loop(0, CH, _row, 0)
        _fill(0.0)
        base = sid * ROWS_PER_SUB
        for r in range(ROWS_PER_SUB // CH):
            pltpu.sync_copy(rows_a, acc.at[pl.ds(base + r * CH, CH)])
        rem = ROWS_PER_SUB % CH
        if rem:
            pltpu.sync_copy(rows_a.at[pl.ds(0, rem)],
                            acc.at[pl.ds(base + (ROWS_PER_SUB // CH) * CH, rem)])
        if not gather:
            _fill(1.0)
        plsc.subcore_barrier()

        def _gst(k, buf, sem):
            pltpu.async_copy(tab_hbm.at[sidx_v.at[k]], buf, sem)

        def _gwt(k, buf, sem):
            pltpu.make_async_copy(tab_hbm.at[sidx_v.at[k]], buf, sem).wait()

        def _sst(k, buf, sem):
            pltpu.async_copy(buf, acc.at[didx_v.at[k]], sem, add=True)

        def _swt(k, buf, sem):
            pltpu.make_async_copy(buf, acc.at[didx_v.at[k]], sem).wait()

        for p in range(n_pass):
            if gather:
                pltpu.sync_copy(
                    sidx_hbm.at[pl.ds(cid * a_s + sid * b_s + p * csub_p,
                                      csub_p)], sidx_v)
            pltpu.sync_copy(
                didx_hbm.at[pl.ds(cid * a_d + sid * b_d + p * csub_p,
                                  csub_p)], didx_v)

            if gather:
                _gst(0, rows_a, semga)

                def _pair(k2, c):
                    a = 2 * k2
                    _gwt(a, rows_a, semga)

                    @pl.when(k2 > 0)
                    def _():
                        _swt(a - 1, rows_b, semsb)
                    _gst(a + 1, rows_b, semgb)
                    _sst(a, rows_a, semsa)
                    _gwt(a + 1, rows_b, semgb)
                    _swt(a, rows_a, semsa)

                    @pl.when(k2 + 1 < npairs)
                    def _():
                        _gst(a + 2, rows_a, semga)
                    _sst(a + 1, rows_b, semsb)
                    return c
                lax.fori_loop(0, npairs, _pair, 0)
                _swt(csub_p - 1, rows_b, semsb)
            else:
                def _fire(k, c):
                    _sst(k, rows_a, semsa)
                    return c
                lax.fori_loop(0, csub_p, _fire, 0)

                def _drain(k, c):
                    _swt(k, rows_a, semsa)
                    return c
                lax.fori_loop(0, csub_p, _drain, 0)
        plsc.subcore_barrier()

        pltpu.sync_copy(
            acc.at[pl.ds(sid * ROWS_PER_SUB, ROWS_PER_SUB)],
            out_hbm.at[pl.ds(cid * NP + sid * ROWS_PER_SUB, ROWS_PER_SUB)])

    scratch = [
        pltpu.VMEM((csub_p, CH), jnp.int32),
        pltpu.VMEM((CH, F), _F32),
        pltpu.VMEM_SHARED((NP, F), _F32),
        pltpu.SemaphoreType.DMA,
    ]
    if gather:
        scratch.insert(0, pltpu.VMEM((csub_p, CH), jnp.int32))
        scratch.insert(3, pltpu.VMEM((CH, F), _F32))
        scratch += [pltpu.SemaphoreType.DMA] * 3
    return pl.kernel(
        body,
        mesh=mesh,
        out_type=jax.ShapeDtypeStruct((2 * NP, F), _F32),
        scratch_types=scratch,
    )


_sc_degrees = _make_sc_agg(128, 160, 0, 0, 2560, 160, gather=False)
_sc_agg128 = _make_sc_agg(128, 80, 1280, 80, 1280, 80, gather=True, n_pass=2)
_sc_agg256 = _make_sc_agg(128, 160, 2560, 160, 0, 160, gather=True, n_pass=4)


R = 2048
GRID = NP // R
_HI = jax.lax.Precision.DEFAULT


def _dot(x, w):
    return jnp.dot(x, w, precision=_HI, preferred_element_type=_F32)


def _b_rows(spec_rows):
    return pl.BlockSpec((spec_rows, 1), lambda i: (i, 0))


_B_P128 = pl.BlockSpec((2, R, 128), lambda i: (0, i, 0))
_B_ROWS128 = pl.BlockSpec((R, 128), lambda i: (i, 0))


def _tc_call(body, in_specs, out_specs, out_shape):
    return pl.pallas_call(
        body, grid=(GRID,), in_specs=in_specs, out_specs=out_specs,
        out_shape=out_shape)


def _rsqrt_body(x_ref, o_ref):
    o_ref[...] = lax.rsqrt(jnp.maximum(x_ref[:, 0:1], 1.0))


def _mm1_body(x_ref, do_ref, w_ref, o_ref):
    o_ref[...] = _dot(do_ref[...] * x_ref[...], w_ref[...])


def _mmsplit_body(p_ref, di_ref, do_ref, b_ref, wa_ref, wb_ref, o_ref):
    u = p_ref[0] + p_ref[1]
    x = do_ref[...] * jax.nn.relu(di_ref[...] * u + b_ref[...])
    o_ref[0] = _dot(x, wa_ref[...])
    o_ref[1] = _dot(x, wb_ref[...])


def _mmh2s_body(u_ref, di_ref, do_ref, ba_ref, bb_ref,
                waa_ref, wba_ref, wab_ref, wbb_ref, o_ref):
    xa = do_ref[...] * jax.nn.relu(di_ref[...] * u_ref[0] + ba_ref[...])
    xb = do_ref[...] * jax.nn.relu(di_ref[...] * u_ref[1] + bb_ref[...])
    o_ref[0] = _dot(xa, waa_ref[...]) + _dot(xb, wba_ref[...])
    o_ref[1] = _dot(xa, wab_ref[...]) + _dot(xb, wbb_ref[...])


def _mm4_body(u_ref, di_ref, do_ref, ba_ref, bb_ref, wa_ref, wb_ref, o_ref):
    xa = do_ref[...] * jax.nn.relu(di_ref[...] * u_ref[0] + ba_ref[...])
    xb = do_ref[...] * jax.nn.relu(di_ref[...] * u_ref[1] + bb_ref[...])
    o_ref[...] = _dot(xa, wa_ref[...]) + _dot(xb, wb_ref[...])


def _mm_body(p_ref, di_ref, do_ref, b_ref, w_ref, o_ref):
    u = p_ref[0] + p_ref[1]
    x = do_ref[...] * jax.nn.relu(di_ref[...] * u + b_ref[...])
    o_ref[...] = _dot(x, w_ref[...])


def _final_body(p_ref, di_ref, b_ref, o_ref):
    s = p_ref[0, :, 0:1] + p_ref[1, :, 0:1]
    o_ref[...] = di_ref[...] * s + b_ref[...]


def kernel(in_feat, edge_index, W1, b1, W2, b2, W3, b3, W4, b4, W5, b5,
           W6, b6, W7, b7):
    src = edge_index[0]
    dst = edge_index[1]
    src_ch = src.reshape(E // CH, CH)
    dst_ch = dst.reshape(E // CH, CH)
    srcdst_ch = jnp.concatenate([src_ch, dst_ch], axis=0)
    srcp_ch = jnp.concatenate([src_ch, src_ch + NP], axis=0)

    x0 = jnp.pad(in_feat, ((0, NP - N), (0, 0)))

    deg2 = _sc_degrees(srcdst_ch)
    d_all = pl.pallas_call(
        _rsqrt_body, out_shape=jax.ShapeDtypeStruct((2 * NP, 1), _F32))(deg2)
    d_o = d_all[:NP]
    d_i = d_all[NP:]

    di_spec = _b_rows(R)
    do_spec = _b_rows(R)
    b128 = pl.BlockSpec((1, 128), lambda i: (0, 0))
    w128 = pl.BlockSpec((128, 128), lambda i: (0, 0))

    v1 = _tc_call(_mm1_body,
                  [_B_ROWS128, do_spec, w128],
                  _B_ROWS128, jax.ShapeDtypeStruct((NP, 128), _F32))(
                      x0, d_o, W1)
    u1 = _sc_agg128(src_ch, dst_ch, v1).reshape(2, NP, 128)

    v2 = _tc_call(_mmsplit_body,
                  [_B_P128, di_spec, do_spec, b128, w128, w128],
                  _B_P128, jax.ShapeDtypeStruct((2, NP, 128), _F32))(
                      u1, d_i, d_o, b1.reshape(1, 128),
                      W2[:, :128], W2[:, 128:])
    u2 = _sc_agg256(srcp_ch, dst_ch, v2.reshape(2 * NP, 128)).reshape(
        2, NP, 128)

    v3 = _tc_call(_mmh2s_body,
                  [_B_P128, di_spec, do_spec, b128, b128,
                   w128, w128, w128, w128],
                  _B_P128, jax.ShapeDtypeStruct((2, NP, 128), _F32))(
                      u2, d_i, d_o, b2[:128].reshape(1, 128),
                      b2[128:].reshape(1, 128),
                      W3[:128, :128], W3[128:, :128],
                      W3[:128, 128:], W3[128:, 128:])
    u3 = _sc_agg256(srcp_ch, dst_ch, v3.reshape(2 * NP, 128)).reshape(
        2, NP, 128)

    v4 = _tc_call(_mm4_body,
                  [_B_P128, di_spec, do_spec, b128, b128, w128, w128],
                  _B_ROWS128, jax.ShapeDtypeStruct((NP, 128), _F32))(
                      u3, d_i, d_o, b3[:128].reshape(1, 128),
                      b3[128:].reshape(1, 128), W4[:128], W4[128:])
    u4 = _sc_agg128(src_ch, dst_ch, v4).reshape(2, NP, 128)

    mm = functools.partial(
        _tc_call, _mm_body,
        [_B_P128, di_spec, do_spec, b128, w128])
    v5 = mm(_B_ROWS128, jax.ShapeDtypeStruct((NP, 128), _F32))(
        u4, d_i, d_o, b4.reshape(1, 128), W5)
    u5 = _sc_agg128(src_ch, dst_ch, v5).reshape(2, NP, 128)
    v6 = mm(_B_ROWS128, jax.ShapeDtypeStruct((NP, 128), _F32))(
        u5, d_i, d_o, b5.reshape(1, 128), W6)
    u6 = _sc_agg128(src_ch, dst_ch, v6).reshape(2, NP, 128)
    w7p = jnp.pad(W7, ((0, 0), (0, 127)))
    v7 = mm(_B_ROWS128, jax.ShapeDtypeStruct((NP, 128), _F32))(
        u6, d_i, d_o, b6.reshape(1, 128), w7p)
    u7 = _sc_agg128(src_ch, dst_ch, v7).reshape(2, NP, 128)

    out = _tc_call(_final_body,
                   [_B_P128, di_spec,
                    pl.BlockSpec((1, 1), lambda i: (0, 0))],
                   pl.BlockSpec((R, 1), lambda i: (i, 0)),
                   jax.ShapeDtypeStruct((NP, 1), _F32))(
                       u7, d_i, b7.reshape(1, 1))
    return out[:N]
